# baseline (speedup 1.0000x reference)
import jax
import jax.numpy as jnp
from jax.experimental import pallas as pl
from jax.experimental.pallas import tpu as pltpu


def _ffn_kernel(x_ref, w1a_ref, w1b_ref, b1_ref, w2a_ref, w2b_ref, b2_ref,
                o_ref):
    CAP = x_ref.shape[1]
    I2 = w1a_ref.shape[2]
    H2 = w2a_ref.shape[2]
    x = x_ref[0]

    def dot_t(a, b):
        return jax.lax.dot_general(
            a, b, (((1,), (1,)), ((), ())), preferred_element_type=jnp.float32
        )

    b1 = b1_ref[0]
    h_a = dot_t(x, w1a_ref[0, 0]) + b1[:, :I2]
    h_b = dot_t(x, w1b_ref[0, 0]) + b1[:, I2:]
    h_a = 0.5 * h_a * (1.0 + jax.lax.erf(h_a * 0.7071067811865476))
    h_b = 0.5 * h_b * (1.0 + jax.lax.erf(h_b * 0.7071067811865476))

    w2a = w2a_ref[0, 0]
    w2b = w2b_ref[0, 0]
    b2 = b2_ref[0]
    y_a = dot_t(h_a, w2a[:, :I2]) + dot_t(h_b, w2a[:, I2:]) + b2[:, :H2]
    y_b = dot_t(h_a, w2b[:, :I2]) + dot_t(h_b, w2b[:, I2:]) + b2[:, H2:]
    o_ref[0] = jnp.concatenate([y_a, y_b], axis=1)


def kernel(inter_state, W1, b1, W2, b2, loads):
    E, I, H = W1.shape
    CAP = inter_state.shape[0] // E

    x = inter_state.reshape(E, CAP, H)
    W1s = W1.reshape(E, 2, I // 2, H)
    W2s = W2.reshape(E, 2, H // 2, I)
    b1r = b1.reshape(E, 1, I)
    b2r = b2.reshape(E, 1, H)

    out = pl.pallas_call(
        _ffn_kernel,
        grid=(E,),
        in_specs=[
            pl.BlockSpec((1, CAP, H), lambda e: (e, 0, 0)),
            pl.BlockSpec((1, 1, I // 2, H), lambda e: (e, 0, 0, 0)),
            pl.BlockSpec((1, 1, I // 2, H), lambda e: (e, 1, 0, 0)),
            pl.BlockSpec((1, 1, I), lambda e: (e, 0, 0)),
            pl.BlockSpec((1, 1, H // 2, I), lambda e: (e, 0, 0, 0)),
            pl.BlockSpec((1, 1, H // 2, I), lambda e: (e, 1, 0, 0)),
            pl.BlockSpec((1, 1, H), lambda e: (e, 0, 0)),
        ],
        out_specs=pl.BlockSpec((1, CAP, H), lambda e: (e, 0, 0)),
        out_shape=jax.ShapeDtypeStruct((E, CAP, H), jnp.float32),
        compiler_params=pltpu.CompilerParams(
            dimension_semantics=("parallel",),
        ),
    )(x, W1s, W1s, b1r, W2s, W2s, b2r)
    return out.reshape(E * CAP, H)
